# Initial kernel scaffold; baseline (speedup 1.0000x reference)
import jax
import jax.numpy as jnp
from jax.experimental import pallas as pl


def kernel(inp, W):
    raise NotImplementedError("write your pallas kernel here")


# speedup vs baseline: 1.0946x; 1.0946x over previous
import jax
import jax.numpy as jnp
from jax.experimental import pallas as pl
from jax.experimental.pallas import tpu as pltpu

S = 2048
D = 4096
E = 64
C = 64
BS = 256
GRID = S // BS


def _gate_block(x_ref, w_ref, cw_ref, dm_ref, carry_ref):
    i = pl.program_id(0)

    @pl.when(i == 0)
    def _():
        carry_ref[...] = jnp.zeros_like(carry_ref)

    x = x_ref[...]
    w = w_ref[...]
    logits = jax.lax.dot_general(
        x, w, (((1,), (1,)), ((), ())),
        preferred_element_type=jnp.float32)

    mx = jnp.max(logits, axis=1, keepdims=True)
    denom = jnp.sum(jnp.exp(logits - mx), axis=1, keepdims=True)
    gate = 1.0 / denom
    eidx = jnp.argmax(logits, axis=1).astype(jnp.int32).reshape(BS, 1)

    ecol = jax.lax.broadcasted_iota(jnp.int32, (BS, E), 1)
    mask = (ecol == eidx).astype(jnp.float32)

    r = jax.lax.broadcasted_iota(jnp.int32, (BS, BS), 0)
    c = jax.lax.broadcasted_iota(jnp.int32, (BS, BS), 1)
    tri = (r >= c).astype(jnp.float32)
    cnt = jax.lax.dot_general(
        tri, mask, (((1,), (0,)), ((), ())),
        preferred_element_type=jnp.float32)

    carry = carry_ref[...]
    locv = cnt - 1.0 + carry
    loc = jnp.sum(locv * mask, axis=1, keepdims=True)
    keep = loc < float(C)
    loci = loc.astype(jnp.int32)
    gk = jnp.where(keep, gate, 0.0)

    carry_ref[...] = carry + jnp.sum(mask, axis=0, keepdims=True)

    col = jax.lax.broadcasted_iota(jnp.int32, (BS, E * C), 1)
    hit = ((col >> 6) == eidx) & ((col & 63) == loci)
    cw_ref[...] = jnp.where(hit, gk, 0.0)
    dm_ref[...] = hit & keep


def kernel(inp, W):
    x = inp.reshape(-1, inp.shape[-1])
    cw2d, dm2d = pl.pallas_call(
        _gate_block,
        grid=(GRID,),
        in_specs=[
            pl.BlockSpec((BS, D), lambda i: (i, 0)),
            pl.BlockSpec((E, D), lambda i: (0, 0)),
        ],
        out_specs=[
            pl.BlockSpec((BS, E * C), lambda i: (i, 0)),
            pl.BlockSpec((BS, E * C), lambda i: (i, 0)),
        ],
        out_shape=[
            jax.ShapeDtypeStruct((S, E * C), jnp.float32),
            jax.ShapeDtypeStruct((S, E * C), jnp.bool_),
        ],
        scratch_shapes=[pltpu.VMEM((1, E), jnp.float32)],
    )(x, W)
    return cw2d.reshape(S, E, C), dm2d.reshape(S, E, C)
